# Initial kernel scaffold; baseline (speedup 1.0000x reference)
import jax
import jax.numpy as jnp
from jax.experimental import pallas as pl


def kernel(x, time_indices, table0, table1, table2, table3, table4, Wp, bp):
    raise NotImplementedError("write your pallas kernel here")


# speedup vs baseline: 1.7574x; 1.7574x over previous
import functools

import jax
import jax.numpy as jnp
from jax import lax
from jax.experimental import pallas as pl
from jax.experimental.pallas import tpu as pltpu
from jax.experimental.pallas import tpu_sc as plsc

B, T = 4, 8192
N_TOK = B * T
D_MODEL = 1024
SPD = 86400
PERIODS = (SPD, SPD // 2, SPD // 3, SPD // 4, SPD // 6)
NT = len(PERIODS)
SUB = 204
SUB_P = 256
LANES = 16
NC, NS = 2, 16
NW = NC * NS
TOK_W = N_TOK // NW
CHUNK = 128
NCHUNK = TOK_W // CHUNK


def _mod_period(v, period):
    out = v
    k = period
    while k < SPD:
        out = out - jnp.where(v >= k, jnp.int32(period), jnp.int32(0))
        k += period
    return out


def _sc_gather(t_flat, tables):
    mesh = plsc.VectorSubcoreMesh(core_axis_name="c", subcore_axis_name="s")
    out_type = tuple(
        jax.ShapeDtypeStruct((N_TOK, SUB_P), jnp.float32) for _ in range(NT))
    scratch = (
        [pltpu.VMEM((TOK_W,), jnp.int32)]
        + [pltpu.VMEM((TOK_W,), jnp.int32) for _ in range(NT)]
        + [pltpu.VMEM((CHUNK, SUB_P), jnp.float32) for _ in range(2)]
        + [pltpu.SemaphoreType.DMA, pltpu.SemaphoreType.DMA]
    )

    @functools.partial(pl.kernel, mesh=mesh, out_type=out_type,
                       scratch_types=scratch)
    def k(t_hbm, tb0, tb1, tb2, tb3, tb4, o0, o1, o2, o3, o4,
          tok_v, i0, i1, i2, i3, i4, buf0, buf1, sem0, sem1):
        wid = lax.axis_index("s") * NC + lax.axis_index("c")
        base = wid * TOK_W
        pltpu.sync_copy(t_hbm.at[pl.ds(base, TOK_W)], tok_v)

        idx = [i0, i1, i2, i3, i4]

        def mod_body(c, carry):
            off = c * LANES
            v = tok_v[pl.ds(off, LANES)]
            for i in range(NT):
                idx[i][pl.ds(off, LANES)] = _mod_period(v, PERIODS[i])
            return carry

        lax.fori_loop(0, TOK_W // LANES, mod_body, 0)

        tbs = [tb0, tb1, tb2, tb3, tb4]
        outs = [o0, o1, o2, o3, o4]
        bufs = [buf0, buf1]
        sems = [sem0, sem1]
        seq = [(i, ch) for i in range(NT) for ch in range(NCHUNK)]
        prev = None
        for j, (i, ch) in enumerate(seq):
            b = bufs[j % 2]
            cp = pltpu.async_copy(
                tbs[i].at[idx[i].at[pl.ds(ch * CHUNK, CHUNK)]], b, sems[j % 2])
            if prev is not None:
                pcp, pb, pdst = prev
                pcp.wait()
                pltpu.sync_copy(pb, pdst)
            prev = (cp, b, outs[i].at[pl.ds(base + ch * CHUNK, CHUNK)])
        pcp, pb, pdst = prev
        pcp.wait()
        pltpu.sync_copy(pb, pdst)

    return k(t_flat, *tables)


def _tc_project(embs, wps, bias):
    BM = 256

    def body(e0, e1, e2, e3, e4, w0, w1, w2, w3, w4, b_ref, o_ref):
        acc = jnp.dot(e0[...], w0[...], preferred_element_type=jnp.float32)
        for e_ref, w_ref in ((e1, w1), (e2, w2), (e3, w3), (e4, w4)):
            acc = acc + jnp.dot(e_ref[...], w_ref[...],
                                preferred_element_type=jnp.float32)
        o_ref[...] = acc + b_ref[...]

    in_specs = (
        [pl.BlockSpec((BM, SUB_P), lambda m: (m, 0)) for _ in range(NT)]
        + [pl.BlockSpec((SUB_P, D_MODEL), lambda m: (0, 0)) for _ in range(NT)]
        + [pl.BlockSpec((1, D_MODEL), lambda m: (0, 0))]
    )
    return pl.pallas_call(
        body,
        grid=(N_TOK // BM,),
        in_specs=in_specs,
        out_specs=pl.BlockSpec((BM, D_MODEL), lambda m: (m, 0)),
        out_shape=jax.ShapeDtypeStruct((N_TOK, D_MODEL), jnp.float32),
    )(*embs, *wps, bias)


def kernel(x, time_indices, table0, table1, table2, table3, table4, Wp, bp):
    del x
    t_flat = time_indices.reshape(N_TOK).astype(jnp.int32)
    pad = ((0, 0), (0, SUB_P - SUB))
    tables = tuple(jnp.pad(t, pad) for t in
                   (table0, table1, table2, table3, table4))
    embs = _sc_gather(t_flat, tables)
    wps = tuple(jnp.pad(Wp[i * SUB:(i + 1) * SUB], ((0, SUB_P - SUB), (0, 0)))
                for i in range(NT))
    out = _tc_project(embs, wps, bp.reshape(1, D_MODEL))
    return out.reshape(B, T, D_MODEL)


# speedup vs baseline: 2.3236x; 1.3222x over previous
import functools

import jax
import jax.numpy as jnp
from jax import lax
from jax.experimental import pallas as pl
from jax.experimental.pallas import tpu as pltpu
from jax.experimental.pallas import tpu_sc as plsc

B, T = 4, 8192
N_TOK = B * T
D_MODEL = 1024
SPD = 86400
PERIODS = (SPD, SPD // 2, SPD // 3, SPD // 4, SPD // 6)
NT = len(PERIODS)
SUB = 204
HEAD = 128
TAIL = SUB - HEAD
LANES = 16
NC, NS = 2, 16
NW = NC * NS
TOK_W = N_TOK // NW
CHUNK = 128
NCHUNK = TOK_W // CHUNK


def _mod_period(v, period):
    out = v
    k = period
    while k < SPD:
        out = out - jnp.where(v >= k, jnp.int32(period), jnp.int32(0))
        k += period
    return out


def _sc_gather(t_flat, tables, tails):
    mesh = plsc.VectorSubcoreMesh(core_axis_name="c", subcore_axis_name="s")
    out_type = tuple(
        jax.ShapeDtypeStruct((N_TOK, HEAD), jnp.float32) for _ in range(2 * NT))
    scratch = (
        [pltpu.VMEM((TOK_W,), jnp.int32)]
        + [pltpu.VMEM((TOK_W,), jnp.int32) for _ in range(NT)]
        + [pltpu.VMEM((CHUNK, HEAD), jnp.float32) for _ in range(4)]
        + [pltpu.SemaphoreType.DMA for _ in range(4)]
    )

    @functools.partial(pl.kernel, mesh=mesh, out_type=out_type,
                       scratch_types=scratch)
    def k(t_hbm, tb0, tb1, tb2, tb3, tb4, tl0, tl1, tl2, tl3, tl4,
          oh0, oh1, oh2, oh3, oh4, ot0, ot1, ot2, ot3, ot4,
          tok_v, i0, i1, i2, i3, i4, bh0, bt0, bh1, bt1,
          smh0, smt0, smh1, smt1):
        wid = lax.axis_index("s") * NC + lax.axis_index("c")
        base = wid * TOK_W
        pltpu.sync_copy(t_hbm.at[pl.ds(base, TOK_W)], tok_v)

        idx = [i0, i1, i2, i3, i4]

        def mod_body(c, carry):
            off = c * LANES
            v = tok_v[pl.ds(off, LANES)]
            for i in range(NT):
                idx[i][pl.ds(off, LANES)] = _mod_period(v, PERIODS[i])
            return carry

        lax.fori_loop(0, TOK_W // LANES, mod_body, 0)

        heads = [tb0, tb1, tb2, tb3, tb4]
        tails_r = [tl0, tl1, tl2, tl3, tl4]
        outs_h = [oh0, oh1, oh2, oh3, oh4]
        outs_t = [ot0, ot1, ot2, ot3, ot4]

        for i in range(NT):
            def pair_body(c, carry, i=i):
                ch0 = c * 2
                ch1 = ch0 + 1
                ix0 = idx[i].at[pl.ds(ch0 * CHUNK, CHUNK)]
                ix1 = idx[i].at[pl.ds(ch1 * CHUNK, CHUNK)]
                h0 = pltpu.async_copy(
                    heads[i].at[ix0, pl.ds(0, HEAD)], bh0, smh0)
                t0 = pltpu.async_copy(tails_r[i].at[ix0], bt0, smt0)
                h1 = pltpu.async_copy(
                    heads[i].at[ix1, pl.ds(0, HEAD)], bh1, smh1)
                t1 = pltpu.async_copy(tails_r[i].at[ix1], bt1, smt1)
                r0 = pl.ds(base + ch0 * CHUNK, CHUNK)
                r1 = pl.ds(base + ch1 * CHUNK, CHUNK)
                h0.wait()
                pltpu.sync_copy(bh0, outs_h[i].at[r0])
                t0.wait()
                pltpu.sync_copy(bt0, outs_t[i].at[r0])
                h1.wait()
                pltpu.sync_copy(bh1, outs_h[i].at[r1])
                t1.wait()
                pltpu.sync_copy(bt1, outs_t[i].at[r1])
                return carry

            lax.fori_loop(0, NCHUNK // 2, pair_body, 0)

    return k(t_flat, *tables, *tails)


def _tc_project(embs, wps, bias):
    BM = 256
    ne = len(embs)

    def body(*refs):
        e_refs = refs[:ne]
        w_refs = refs[ne:2 * ne]
        b_ref = refs[2 * ne]
        o_ref = refs[2 * ne + 1]
        acc = b_ref[...]
        for e_ref, w_ref in zip(e_refs, w_refs):
            acc = acc + jnp.dot(e_ref[...], w_ref[...],
                                preferred_element_type=jnp.float32)
        o_ref[...] = acc

    in_specs = (
        [pl.BlockSpec((BM, HEAD), lambda m: (m, 0)) for _ in range(ne)]
        + [pl.BlockSpec((HEAD, D_MODEL), lambda m: (0, 0)) for _ in range(ne)]
        + [pl.BlockSpec((1, D_MODEL), lambda m: (0, 0))]
    )
    return pl.pallas_call(
        body,
        grid=(N_TOK // BM,),
        in_specs=in_specs,
        out_specs=pl.BlockSpec((BM, D_MODEL), lambda m: (m, 0)),
        out_shape=jax.ShapeDtypeStruct((N_TOK, D_MODEL), jnp.float32),
    )(*embs, *wps, bias)


def kernel(x, time_indices, table0, table1, table2, table3, table4, Wp, bp):
    del x
    t_flat = time_indices.reshape(N_TOK).astype(jnp.int32)
    tables = (table0, table1, table2, table3, table4)
    tails = tuple(jnp.pad(t[:, HEAD:], ((0, 0), (0, HEAD - TAIL)))
                  for t in tables)
    embs = _sc_gather(t_flat, tables, tails)
    wh = tuple(Wp[i * SUB:i * SUB + HEAD] for i in range(NT))
    wt = tuple(jnp.pad(Wp[i * SUB + HEAD:(i + 1) * SUB],
                       ((0, HEAD - TAIL), (0, 0))) for i in range(NT))
    out = _tc_project(embs, wh + wt, bp.reshape(1, D_MODEL))
    return out.reshape(B, T, D_MODEL)


# speedup vs baseline: 2.3282x; 1.0020x over previous
import functools

import jax
import jax.numpy as jnp
from jax import lax
from jax.experimental import pallas as pl
from jax.experimental.pallas import tpu as pltpu
from jax.experimental.pallas import tpu_sc as plsc

B, T = 4, 8192
N_TOK = B * T
D_MODEL = 1024
SPD = 86400
PERIODS = (SPD, SPD // 2, SPD // 3, SPD // 4, SPD // 6)
NT = len(PERIODS)
SUB = 204
HEAD = 128
TAIL = SUB - HEAD
LANES = 16
NC, NS = 2, 16
NW = NC * NS
TOK_W = N_TOK // NW
CHUNK = 128
NCHUNK = TOK_W // CHUNK


def _mod_period(v, period):
    out = v
    k = period
    while k < SPD:
        out = out - jnp.where(v >= k, jnp.int32(period), jnp.int32(0))
        k += period
    return out


def _sc_gather(t_flat, tables, tails):
    mesh = plsc.VectorSubcoreMesh(core_axis_name="c", subcore_axis_name="s")
    out_type = tuple(
        jax.ShapeDtypeStruct((N_TOK, HEAD), jnp.float32) for _ in range(2 * NT))
    scratch = (
        [pltpu.VMEM((TOK_W,), jnp.int32)]
        + [pltpu.VMEM((TOK_W,), jnp.int32) for _ in range(NT)]
        + [pltpu.VMEM((CHUNK, HEAD), jnp.float32) for _ in range(4)]
        + [pltpu.SemaphoreType.DMA for _ in range(4)]
    )

    @functools.partial(pl.kernel, mesh=mesh, out_type=out_type,
                       scratch_types=scratch)
    def k(t_hbm, tb0, tb1, tb2, tb3, tb4, tl0, tl1, tl2, tl3, tl4,
          oh0, oh1, oh2, oh3, oh4, ot0, ot1, ot2, ot3, ot4,
          tok_v, i0, i1, i2, i3, i4, bh0, bt0, bh1, bt1,
          smh0, smt0, smh1, smt1):
        wid = lax.axis_index("s") * NC + lax.axis_index("c")
        base = wid * TOK_W
        pltpu.sync_copy(t_hbm.at[pl.ds(base, TOK_W)], tok_v)

        idx = [i0, i1, i2, i3, i4]

        def mod_body(c, carry):
            off = c * LANES
            v = tok_v[pl.ds(off, LANES)]
            for i in range(NT):
                idx[i][pl.ds(off, LANES)] = _mod_period(v, PERIODS[i])
            return carry

        lax.fori_loop(0, TOK_W // LANES, mod_body, 0)

        heads = [tb0, tb1, tb2, tb3, tb4]
        tails_r = [tl0, tl1, tl2, tl3, tl4]
        outs_h = [oh0, oh1, oh2, oh3, oh4]
        outs_t = [ot0, ot1, ot2, ot3, ot4]

        for i in range(NT):
            def pair_body(c, carry, i=i):
                ch0 = c * 2
                ch1 = ch0 + 1
                ix0 = idx[i].at[pl.ds(ch0 * CHUNK, CHUNK)]
                ix1 = idx[i].at[pl.ds(ch1 * CHUNK, CHUNK)]
                h0 = pltpu.async_copy(
                    heads[i].at[ix0, pl.ds(0, HEAD)], bh0, smh0)
                t0 = pltpu.async_copy(tails_r[i].at[ix0], bt0, smt0)
                h1 = pltpu.async_copy(
                    heads[i].at[ix1, pl.ds(0, HEAD)], bh1, smh1)
                t1 = pltpu.async_copy(tails_r[i].at[ix1], bt1, smt1)
                r0 = pl.ds(base + ch0 * CHUNK, CHUNK)
                r1 = pl.ds(base + ch1 * CHUNK, CHUNK)
                h0.wait()
                pltpu.sync_copy(bh0, outs_h[i].at[r0])
                t0.wait()
                pltpu.sync_copy(bt0, outs_t[i].at[r0])
                h1.wait()
                pltpu.sync_copy(bh1, outs_h[i].at[r1])
                t1.wait()
                pltpu.sync_copy(bt1, outs_t[i].at[r1])
                return carry

            lax.fori_loop(0, NCHUNK // 2, pair_body, 0)

    return k(t_flat, *tables, *tails)


def _tc_project(embs, wps, bias):
    BM = 256
    ne = len(embs)

    def body(*refs):
        e_refs = refs[:ne]
        w_refs = refs[ne:2 * ne]
        b_ref = refs[2 * ne]
        o_ref = refs[2 * ne + 1]
        acc = b_ref[...]
        for e_ref, w_ref in zip(e_refs, w_refs):
            acc = acc + jnp.dot(e_ref[...].astype(jnp.bfloat16), w_ref[...],
                                preferred_element_type=jnp.float32)
        o_ref[...] = acc

    in_specs = (
        [pl.BlockSpec((BM, HEAD), lambda m: (m, 0)) for _ in range(ne)]
        + [pl.BlockSpec((HEAD, D_MODEL), lambda m: (0, 0)) for _ in range(ne)]
        + [pl.BlockSpec((1, D_MODEL), lambda m: (0, 0))]
    )
    wps = tuple(w.astype(jnp.bfloat16) for w in wps)
    return pl.pallas_call(
        body,
        grid=(N_TOK // BM,),
        in_specs=in_specs,
        out_specs=pl.BlockSpec((BM, D_MODEL), lambda m: (m, 0)),
        out_shape=jax.ShapeDtypeStruct((N_TOK, D_MODEL), jnp.float32),
    )(*embs, *wps, bias)


def kernel(x, time_indices, table0, table1, table2, table3, table4, Wp, bp):
    del x
    t_flat = time_indices.reshape(N_TOK).astype(jnp.int32)
    tables = (table0, table1, table2, table3, table4)
    tails = tuple(jnp.pad(t[:, HEAD:], ((0, 0), (0, HEAD - TAIL)))
                  for t in tables)
    embs = _sc_gather(t_flat, tables, tails)
    wh = tuple(Wp[i * SUB:i * SUB + HEAD] for i in range(NT))
    wt = tuple(jnp.pad(Wp[i * SUB + HEAD:(i + 1) * SUB],
                       ((0, HEAD - TAIL), (0, 0))) for i in range(NT))
    out = _tc_project(embs, wh + wt, bp.reshape(1, D_MODEL))
    return out.reshape(B, T, D_MODEL)


# speedup vs baseline: 2.4918x; 1.0702x over previous
import functools

import jax
import jax.numpy as jnp
from jax import lax
from jax.experimental import pallas as pl
from jax.experimental.pallas import tpu as pltpu
from jax.experimental.pallas import tpu_sc as plsc

B, T = 4, 8192
N_TOK = B * T
D_MODEL = 1024
SPD = 86400
PERIODS = (SPD, SPD // 2, SPD // 3, SPD // 4, SPD // 6)
NT = len(PERIODS)
SUB = 204
HEAD = 128
TAIL = SUB - HEAD
SEG = 2 * HEAD
K_TOT = NT * SEG
LANES = 16
NC, NS = 2, 16
NW = NC * NS
TOK_W = N_TOK // NW
CHUNK = 128
NCHUNK = TOK_W // CHUNK


def _mod_period(v, period):
    out = v
    k = period
    while k < SPD:
        out = out - jnp.where(v >= k, jnp.int32(period), jnp.int32(0))
        k += period
    return out


def _sc_gather(t_flat, tables, tails):
    mesh = plsc.VectorSubcoreMesh(core_axis_name="c", subcore_axis_name="s")
    out_type = jax.ShapeDtypeStruct((N_TOK, K_TOT), jnp.float32)
    scratch = (
        [pltpu.VMEM((TOK_W,), jnp.int32)]
        + [pltpu.VMEM((TOK_W,), jnp.int32) for _ in range(NT)]
        + [pltpu.VMEM((CHUNK, HEAD), jnp.float32) for _ in range(4)]
        + [pltpu.SemaphoreType.DMA for _ in range(4)]
    )

    @functools.partial(pl.kernel, mesh=mesh, out_type=out_type,
                       scratch_types=scratch)
    def k(t_hbm, tb0, tb1, tb2, tb3, tb4, tl0, tl1, tl2, tl3, tl4, out,
          tok_v, i0, i1, i2, i3, i4, bh0, bt0, bh1, bt1,
          smh0, smt0, smh1, smt1):
        wid = lax.axis_index("s") * NC + lax.axis_index("c")
        base = wid * TOK_W
        pltpu.sync_copy(t_hbm.at[pl.ds(base, TOK_W)], tok_v)

        idx = [i0, i1, i2, i3, i4]

        def mod_body(c, carry):
            off = c * LANES
            v = tok_v[pl.ds(off, LANES)]
            for i in range(NT):
                idx[i][pl.ds(off, LANES)] = _mod_period(v, PERIODS[i])
            return carry

        lax.fori_loop(0, TOK_W // LANES, mod_body, 0)

        heads = [tb0, tb1, tb2, tb3, tb4]
        tails_r = [tl0, tl1, tl2, tl3, tl4]

        for i in range(NT):
            ch_col = pl.ds(i * SEG, HEAD)
            ct_col = pl.ds(i * SEG + HEAD, HEAD)

            def pair_body(c, carry, i=i, ch_col=ch_col, ct_col=ct_col):
                ch0 = c * 2
                ch1 = ch0 + 1
                ix0 = idx[i].at[pl.ds(ch0 * CHUNK, CHUNK)]
                ix1 = idx[i].at[pl.ds(ch1 * CHUNK, CHUNK)]
                h0 = pltpu.async_copy(
                    heads[i].at[ix0, pl.ds(0, HEAD)], bh0, smh0)
                t0 = pltpu.async_copy(tails_r[i].at[ix0], bt0, smt0)
                h1 = pltpu.async_copy(
                    heads[i].at[ix1, pl.ds(0, HEAD)], bh1, smh1)
                t1 = pltpu.async_copy(tails_r[i].at[ix1], bt1, smt1)
                r0 = pl.ds(base + ch0 * CHUNK, CHUNK)
                r1 = pl.ds(base + ch1 * CHUNK, CHUNK)
                h0.wait()
                pltpu.sync_copy(bh0, out.at[r0, ch_col])
                t0.wait()
                pltpu.sync_copy(bt0, out.at[r0, ct_col])
                h1.wait()
                pltpu.sync_copy(bh1, out.at[r1, ch_col])
                t1.wait()
                pltpu.sync_copy(bt1, out.at[r1, ct_col])
                return carry

            lax.fori_loop(0, NCHUNK // 2, pair_body, 0)

    return k(t_flat, *tables, *tails)


def _tc_project(emb, w, bias):
    BM = 256

    def body(e_ref, w_ref, b_ref, o_ref):
        o_ref[...] = b_ref[...] + jnp.dot(
            e_ref[...].astype(jnp.bfloat16), w_ref[...],
            preferred_element_type=jnp.float32)

    return pl.pallas_call(
        body,
        grid=(N_TOK // BM,),
        in_specs=[
            pl.BlockSpec((BM, K_TOT), lambda m: (m, 0)),
            pl.BlockSpec((K_TOT, D_MODEL), lambda m: (0, 0)),
            pl.BlockSpec((1, D_MODEL), lambda m: (0, 0)),
        ],
        out_specs=pl.BlockSpec((BM, D_MODEL), lambda m: (m, 0)),
        out_shape=jax.ShapeDtypeStruct((N_TOK, D_MODEL), jnp.float32),
    )(emb, w, bias)


def kernel(x, time_indices, table0, table1, table2, table3, table4, Wp, bp):
    del x
    t_flat = time_indices.reshape(N_TOK).astype(jnp.int32)
    tables = (table0, table1, table2, table3, table4)
    tails = tuple(jnp.pad(t[:, HEAD:], ((0, 0), (0, HEAD - TAIL)))
                  for t in tables)
    emb = _sc_gather(t_flat, tables, tails)
    zrows = jnp.zeros((SEG - SUB, D_MODEL), jnp.float32)
    w = jnp.concatenate(
        [p for i in range(NT)
         for p in (Wp[i * SUB:(i + 1) * SUB], zrows)]).astype(jnp.bfloat16)
    out = _tc_project(emb, w, bp.reshape(1, D_MODEL))
    return out.reshape(B, T, D_MODEL)


# speedup vs baseline: 2.5039x; 1.0049x over previous
import functools

import jax
import jax.numpy as jnp
from jax import lax
from jax.experimental import pallas as pl
from jax.experimental.pallas import tpu as pltpu
from jax.experimental.pallas import tpu_sc as plsc

B, T = 4, 8192
N_TOK = B * T
D_MODEL = 1024
SPD = 86400
PERIODS = (SPD, SPD // 2, SPD // 3, SPD // 4, SPD // 6)
NT = len(PERIODS)
SUB = 204
HEAD = 128
TAIL = SUB - HEAD
SEG = 2 * HEAD
K_TOT = NT * SEG
LANES = 16
NC, NS = 2, 16
NW = NC * NS
TOK_W = N_TOK // NW
CHUNK = 128
NCHUNK = TOK_W // CHUNK


def _mod_period(v, period):
    out = v
    k = period
    while k < SPD:
        out = out - jnp.where(v >= k, jnp.int32(period), jnp.int32(0))
        k += period
    return out


def _sc_gather(t_flat, tables, tails):
    mesh = plsc.VectorSubcoreMesh(core_axis_name="c", subcore_axis_name="s")
    out_type = jax.ShapeDtypeStruct((N_TOK, K_TOT), jnp.float32)
    scratch = (
        [pltpu.VMEM((TOK_W,), jnp.int32)]
        + [pltpu.VMEM((TOK_W,), jnp.int32) for _ in range(NT)]
        + [pltpu.VMEM((CHUNK, HEAD), jnp.float32) for _ in range(4)]
        + [pltpu.SemaphoreType.DMA for _ in range(4)]
    )

    @functools.partial(pl.kernel, mesh=mesh, out_type=out_type,
                       scratch_types=scratch,
                       compiler_params=pltpu.CompilerParams(
                           use_tc_tiling_on_sc=True))
    def k(t_hbm, tb0, tb1, tb2, tb3, tb4, tl0, tl1, tl2, tl3, tl4, out,
          tok_v, i0, i1, i2, i3, i4, bh0, bt0, bh1, bt1,
          smh0, smt0, smh1, smt1):
        wid = lax.axis_index("s") * NC + lax.axis_index("c")
        base = wid * TOK_W
        pltpu.sync_copy(t_hbm.at[pl.ds(base, TOK_W)], tok_v)

        idx = [i0, i1, i2, i3, i4]

        def mod_body(c, carry):
            off = c * LANES
            v = tok_v[pl.ds(off, LANES)]
            for i in range(NT):
                idx[i][pl.ds(off, LANES)] = _mod_period(v, PERIODS[i])
            return carry

        lax.fori_loop(0, TOK_W // LANES, mod_body, 0)

        heads = [tb0, tb1, tb2, tb3, tb4]
        tails_r = [tl0, tl1, tl2, tl3, tl4]

        for i in range(NT):
            ch_col = pl.ds(i * SEG, HEAD)
            ct_col = pl.ds(i * SEG + HEAD, HEAD)

            def pair_body(c, carry, i=i, ch_col=ch_col, ct_col=ct_col):
                ch0 = c * 2
                ch1 = ch0 + 1
                ix0 = idx[i].at[pl.ds(ch0 * CHUNK, CHUNK)]
                ix1 = idx[i].at[pl.ds(ch1 * CHUNK, CHUNK)]
                h0 = pltpu.async_copy(
                    heads[i].at[ix0, pl.ds(0, HEAD)], bh0, smh0)
                t0 = pltpu.async_copy(tails_r[i].at[ix0], bt0, smt0)
                h1 = pltpu.async_copy(
                    heads[i].at[ix1, pl.ds(0, HEAD)], bh1, smh1)
                t1 = pltpu.async_copy(tails_r[i].at[ix1], bt1, smt1)
                r0 = pl.ds(base + ch0 * CHUNK, CHUNK)
                r1 = pl.ds(base + ch1 * CHUNK, CHUNK)
                h0.wait()
                pltpu.sync_copy(bh0, out.at[r0, ch_col])
                t0.wait()
                pltpu.sync_copy(bt0, out.at[r0, ct_col])
                h1.wait()
                pltpu.sync_copy(bh1, out.at[r1, ch_col])
                t1.wait()
                pltpu.sync_copy(bt1, out.at[r1, ct_col])
                return carry

            lax.fori_loop(0, NCHUNK // 2, pair_body, 0)

    return k(t_flat, *tables, *tails)


def _tc_project(emb, w, bias):
    BM = 256

    def body(e_ref, w_ref, b_ref, o_ref):
        o_ref[...] = b_ref[...] + jnp.dot(
            e_ref[...].astype(jnp.bfloat16), w_ref[...],
            preferred_element_type=jnp.float32)

    return pl.pallas_call(
        body,
        grid=(N_TOK // BM,),
        in_specs=[
            pl.BlockSpec((BM, K_TOT), lambda m: (m, 0)),
            pl.BlockSpec((K_TOT, D_MODEL), lambda m: (0, 0)),
            pl.BlockSpec((1, D_MODEL), lambda m: (0, 0)),
        ],
        out_specs=pl.BlockSpec((BM, D_MODEL), lambda m: (m, 0)),
        out_shape=jax.ShapeDtypeStruct((N_TOK, D_MODEL), jnp.float32),
    )(emb, w, bias)


def kernel(x, time_indices, table0, table1, table2, table3, table4, Wp, bp):
    del x
    t_flat = time_indices.reshape(N_TOK).astype(jnp.int32)
    tables = (table0, table1, table2, table3, table4)
    tails = tuple(jnp.pad(t[:, HEAD:], ((0, 0), (0, HEAD - TAIL)))
                  for t in tables)
    emb = _sc_gather(t_flat, tables, tails)
    zrows = jnp.zeros((SEG - SUB, D_MODEL), jnp.float32)
    w = jnp.concatenate(
        [p for i in range(NT)
         for p in (Wp[i * SUB:(i + 1) * SUB], zrows)]).astype(jnp.bfloat16)
    out = _tc_project(emb, w, bp.reshape(1, D_MODEL))
    return out.reshape(B, T, D_MODEL)
